# baseline (speedup 1.0000x reference)
import jax
import jax.numpy as jnp
from jax.experimental import pallas as pl
from jax.experimental.pallas import tpu as pltpu

_B = 128
_J = 17
_H0 = 48
_W0 = 48
_HT = 96
_WT = 96
_C = 8


def _fiota(shape, dim):
    return jax.lax.broadcasted_iota(jnp.int32, shape, dim).astype(jnp.float32)


def _up_x(s):
    sm1 = jnp.concatenate([s[..., :1], s[..., :-1]], axis=-1)
    sp1 = jnp.concatenate([s[..., 1:], s[..., -1:]], axis=-1)
    lane = jax.lax.broadcasted_iota(jnp.int32, (1, 1, 1, _W0), 3)
    ex = jnp.where(lane == 0, s, 0.25 * sm1 + 0.75 * s)
    ox = jnp.where(lane == _W0 - 1, s, 0.75 * s + 0.25 * sp1)
    return jnp.concatenate([ex, ox], axis=-1)


def _up_y(a):
    am1 = jnp.concatenate([a[:, :, :1, :], a[:, :, :-1, :]], axis=2)
    ap1 = jnp.concatenate([a[:, :, 1:, :], a[:, :, -1:, :]], axis=2)
    row = jax.lax.broadcasted_iota(jnp.int32, (1, 1, _H0, 1), 2)
    ey = jnp.where(row == 0, a, 0.25 * am1 + 0.75 * a)
    oy = jnp.where(row == _H0 - 1, a, 0.75 * a + 0.25 * ap1)
    return ey, oy


def _body(hm_ref, ct_ref, rg_ref, of_ref, out_ref):
    f32 = jnp.float32
    i32 = jnp.int32
    big = _HT * _WT

    a18 = jnp.concatenate([ct_ref[...], hm_ref[...]], axis=1)

    a_x = _up_x(a18)
    ey, oy = _up_y(a_x)

    lane_l = jax.lax.broadcasted_iota(i32, (1, _HT), 1)
    x_of_l = 2 * (lane_l % _W0) + lane_l // _W0
    row_i = jax.lax.broadcasted_iota(i32, (_H0, 1), 0)
    lin_e = ((2 * row_i) * _WT + x_of_l).reshape(1, 1, _H0, _HT)
    lin_o = ((2 * row_i + 1) * _WT + x_of_l).reshape(1, 1, _H0, _HT)
    xq = x_of_l.astype(f32).reshape(1, 1, 1, _HT)
    yq_e = (2 * row_i).astype(f32).reshape(1, 1, _H0, 1)
    yq_o = (2 * row_i + 1).astype(f32).reshape(1, 1, _H0, 1)

    ct_e = ey[:, 0:1]
    ct_o = oy[:, 0:1]
    m_ct = jnp.maximum(
        jnp.max(jnp.max(ct_e, axis=3, keepdims=True), axis=2, keepdims=True),
        jnp.max(jnp.max(ct_o, axis=3, keepdims=True), axis=2, keepdims=True))
    ids_e = jnp.min(jnp.min(jnp.where(ct_e == m_ct, lin_e, big),
                            axis=3, keepdims=True), axis=2, keepdims=True)
    ids_o = jnp.min(jnp.min(jnp.where(ct_o == m_ct, lin_o, big),
                            axis=3, keepdims=True), axis=2, keepdims=True)
    ids = jnp.minimum(ids_e, ids_o).reshape(_C, 1, 1)
    cy = ids // _WT
    cx = ids % _WT

    sy = ((cy.astype(f32) + 0.5) * 0.5 - 0.5).reshape(_C, 1, 1, 1, 1)
    sx = ((cx.astype(f32) + 0.5) * 0.5 - 0.5).reshape(_C, 1, 1, 1)
    y_i = _fiota((1, 1, 1, _H0, 1), 3)
    w_y = jnp.maximum(0.0, 1.0 - jnp.abs(y_i - sy))
    w_y = w_y / jnp.sum(w_y, axis=3, keepdims=True)
    x_i = _fiota((1, 1, 1, _W0), 3)
    w_x = jnp.maximum(0.0, 1.0 - jnp.abs(x_i - sx))
    w_x = w_x / jnp.sum(w_x, axis=3, keepdims=True)

    rg0 = rg_ref[...]
    rg_v = jnp.sum(jnp.sum(rg0 * w_y, axis=3) * w_x, axis=3)
    reg_x = jnp.clip(cx.reshape(_C, 1).astype(f32) + rg_v[:, :, 0] + 0.5,
                     0.0, _WT - 1.0)
    reg_y = jnp.clip(cy.reshape(_C, 1).astype(f32) + rg_v[:, :, 1] + 0.5,
                     0.0, _HT - 1.0)

    rx = reg_x.reshape(_C, _J, 1, 1)
    ry = reg_y.reshape(_C, _J, 1, 1)
    hm_e = ey[:, 1:]
    hm_o = oy[:, 1:]
    d2_e = (xq - rx) ** 2 + (yq_e - ry) ** 2
    d2_o = (xq - rx) ** 2 + (yq_o - ry) ** 2
    tmp_e = hm_e / jnp.sqrt(d2_e + 1e-9) / 1.8
    tmp_o = hm_o / jnp.sqrt(d2_o + 1e-9) / 1.8
    m2 = jnp.maximum(
        jnp.max(jnp.max(tmp_e, axis=3, keepdims=True), axis=2, keepdims=True),
        jnp.max(jnp.max(tmp_o, axis=3, keepdims=True), axis=2, keepdims=True))
    ids2_e = jnp.min(jnp.min(jnp.where(tmp_e == m2, lin_e, big),
                             axis=3, keepdims=True), axis=2, keepdims=True)
    ids2_o = jnp.min(jnp.min(jnp.where(tmp_o == m2, lin_o, big),
                             axis=3, keepdims=True), axis=2, keepdims=True)
    ids2 = jnp.minimum(ids2_e, ids2_o)
    jy = ids2 // _WT
    jx = ids2 % _WT

    d2_at = (jx.astype(f32) - rx) ** 2 + (jy.astype(f32) - ry) ** 2
    score = m2 * 1.8 * jnp.sqrt(d2_at + 1e-9)

    sy2 = ((jy.astype(f32) + 0.5) * 0.5 - 0.5).reshape(_C, _J, 1, 1, 1)
    sx2 = ((jx.astype(f32) + 0.5) * 0.5 - 0.5).reshape(_C, _J, 1, 1)
    y_i5 = _fiota((1, 1, 1, _H0, 1), 3)
    w_y2 = jnp.maximum(0.0, 1.0 - jnp.abs(y_i5 - sy2))
    w_y2 = w_y2 / jnp.sum(w_y2, axis=3, keepdims=True)
    x_i4 = _fiota((1, 1, 1, _W0), 3)
    w_x2 = jnp.maximum(0.0, 1.0 - jnp.abs(x_i4 - sx2))
    w_x2 = w_x2 / jnp.sum(w_x2, axis=3, keepdims=True)

    of0 = of_ref[...]
    of_v = jnp.sum(jnp.sum(of0 * w_y2, axis=3) * w_x2, axis=3)

    x_norm = (jx.reshape(_C, _J, 1).astype(f32) + of_v[:, :, 0:1]) / float(_WT)
    y_norm = (jy.reshape(_C, _J, 1).astype(f32) + of_v[:, :, 1:2]) / float(_HT)
    out = jnp.concatenate([x_norm, y_norm, score.reshape(_C, _J, 1)], axis=2)
    out_ref[...] = out


def kernel(hm, ct, rg, of):
    rg5 = rg.reshape(_B, _J, 2, _H0, _W0)
    of5 = of.reshape(_B, _J, 2, _H0, _W0)
    out = pl.pallas_call(
        _body,
        grid=(_B // _C,),
        in_specs=[
            pl.BlockSpec((_C, _J, _H0, _W0), lambda b: (b, 0, 0, 0)),
            pl.BlockSpec((_C, 1, _H0, _W0), lambda b: (b, 0, 0, 0)),
            pl.BlockSpec((_C, _J, 2, _H0, _W0), lambda b: (b, 0, 0, 0, 0)),
            pl.BlockSpec((_C, _J, 2, _H0, _W0), lambda b: (b, 0, 0, 0, 0)),
        ],
        out_specs=pl.BlockSpec((_C, _J, 3), lambda b: (b, 0, 0)),
        out_shape=jax.ShapeDtypeStruct((_B, _J, 3), jnp.float32),
        compiler_params=pltpu.CompilerParams(
            dimension_semantics=("parallel",),
        ),
    )(hm, ct, rg5, of5)
    return out.reshape(_B, 3 * _J)
